# Initial kernel scaffold; baseline (speedup 1.0000x reference)
import jax
import jax.numpy as jnp
from jax.experimental import pallas as pl


def kernel(hidden_states, diffusion_timestep, diffusion_token_state, position_ids, W_router, ts_bias, st_bias, pos_bias, router_bias, routed_gate, routed_up, routed_down, shared_gate, shared_up, shared_down):
    raise NotImplementedError("write your pallas kernel here")


# speedup vs baseline: 1.0482x; 1.0482x over previous
import functools

import jax
import jax.numpy as jnp
from jax.experimental import pallas as pl

F32 = jnp.float32
BF16 = jnp.bfloat16


def _router_body(x_ref, wr_ref, base_ref, st_ref, posb_ref, tok_ref, pos_ref,
                 comb_ref, *, S, E, NB):
    logits = jax.lax.dot_general(
        x_ref[...].astype(BF16), wr_ref[...].astype(BF16),
        (((1,), (1,)), ((), ())), preferred_element_type=F32)
    scores = jax.nn.sigmoid(logits)
    tok = tok_ref[...]
    pos = pos_ref[...]
    bias_st = jnp.where(tok == 0, st_ref[0:1, :E], st_ref[1:2, :E])
    onehot = (jax.lax.broadcasted_iota(jnp.int32, (S, NB), 1) == pos).astype(F32)
    bias_pos = jnp.dot(onehot, posb_ref[...], preferred_element_type=F32,
                       precision=jax.lax.Precision.HIGHEST)
    gating = scores + base_ref[...] + bias_st + bias_pos

    lane = jax.lax.broadcasted_iota(jnp.int32, (S, E), 1)
    m1 = jnp.max(gating, axis=1, keepdims=True)
    idx1 = jnp.min(jnp.where(gating == m1, lane, E), axis=1, keepdims=True)
    g2 = jnp.where(lane == idx1, -jnp.inf, gating)
    m2 = jnp.max(g2, axis=1, keepdims=True)
    idx2 = jnp.min(jnp.where(g2 == m2, lane, E), axis=1, keepdims=True)
    s1 = jnp.sum(jnp.where(lane == idx1, scores, 0.0), axis=1, keepdims=True)
    s2 = jnp.sum(jnp.where(lane == idx2, scores, 0.0), axis=1, keepdims=True)
    denom = s1 + s2 + jnp.float32(1e-9)
    comb_ref[...] = (jnp.where(lane == idx1, s1 / denom, 0.0)
                     + jnp.where(lane == idx2, s2 / denom, 0.0))


def _routed_body(xb_ref, comb_ref, g_ref, u_ref, d_ref, out_ref, *, S, E):
    e = pl.program_id(0)
    hb = pl.program_id(1)
    lane = jax.lax.broadcasted_iota(jnp.int32, (S, E), 1)
    c = jnp.sum(jnp.where(lane == e, comb_ref[...], 0.0), axis=1, keepdims=True)
    xb = xb_ref[...]
    g = jnp.dot(xb, g_ref[0], preferred_element_type=F32)
    u = jnp.dot(xb, u_ref[0], preferred_element_type=F32)
    hc = ((jax.nn.silu(g) * u) * c).astype(BF16)
    y = jnp.dot(hc, d_ref[0], preferred_element_type=F32)

    @pl.when((e == 0) & (hb == 0))
    def _():
        out_ref[...] = y

    @pl.when((e != 0) | (hb != 0))
    def _():
        out_ref[...] += y


def _shared_body(xb_ref, routed_ref, g_ref, u_ref, d_ref, out_ref):
    hb = pl.program_id(1)
    xb = xb_ref[...]
    g = jnp.dot(xb, g_ref[...], preferred_element_type=F32)
    u = jnp.dot(xb, u_ref[...], preferred_element_type=F32)
    h = (jax.nn.silu(g) * u).astype(BF16)
    y = jnp.dot(h, d_ref[...], preferred_element_type=F32)

    @pl.when(hb == 0)
    def _():
        out_ref[...] = routed_ref[...] + y

    @pl.when(hb != 0)
    def _():
        out_ref[...] += y


def kernel(hidden_states, diffusion_timestep, diffusion_token_state,
           position_ids, W_router, ts_bias, st_bias, pos_bias, router_bias,
           routed_gate, routed_up, routed_down,
           shared_gate, shared_up, shared_down):
    B, S, D = hidden_states.shape
    E, _, H = routed_gate.shape
    NB = pos_bias.shape[0]
    Hs = shared_gate.shape[1]

    x = hidden_states.reshape(S, D)
    tok = diffusion_token_state.reshape(S, 1).astype(jnp.int32)
    pos = jnp.clip(position_ids.reshape(S, 1).astype(jnp.int32), 0, NB - 1)
    base = (router_bias + ts_bias[diffusion_timestep[0]]).reshape(1, E)
    st_pad = jnp.zeros((8, E), F32).at[:2].set(st_bias)

    comb = pl.pallas_call(
        functools.partial(_router_body, S=S, E=E, NB=NB),
        out_shape=jax.ShapeDtypeStruct((S, E), F32),
    )(x, W_router, base, st_pad, pos_bias, tok, pos)

    xb = x.astype(BF16)
    gb = routed_gate.astype(BF16)
    ub = routed_up.astype(BF16)
    db = routed_down.astype(BF16)

    RHB = 4
    RHblk = H // RHB
    routed = pl.pallas_call(
        functools.partial(_routed_body, S=S, E=E),
        grid=(E, RHB),
        in_specs=[
            pl.BlockSpec((S, D), lambda e, h: (0, 0)),
            pl.BlockSpec((S, E), lambda e, h: (0, 0)),
            pl.BlockSpec((1, D, RHblk), lambda e, h: (e, 0, h)),
            pl.BlockSpec((1, D, RHblk), lambda e, h: (e, 0, h)),
            pl.BlockSpec((1, RHblk, D), lambda e, h: (e, h, 0)),
        ],
        out_specs=pl.BlockSpec((S, D), lambda e, h: (0, 0)),
        out_shape=jax.ShapeDtypeStruct((S, D), F32),
    )(xb, comb, gb, ub, db)

    TB = 4
    Sblk = S // TB
    HB = 4
    Hblk = Hs // HB
    sgb = shared_gate.astype(BF16)
    sub = shared_up.astype(BF16)
    sdb = shared_down.astype(BF16)
    out = pl.pallas_call(
        _shared_body,
        grid=(TB, HB),
        in_specs=[
            pl.BlockSpec((Sblk, D), lambda t, h: (t, 0)),
            pl.BlockSpec((Sblk, D), lambda t, h: (t, 0)),
            pl.BlockSpec((D, Hblk), lambda t, h: (0, h)),
            pl.BlockSpec((D, Hblk), lambda t, h: (0, h)),
            pl.BlockSpec((Hblk, D), lambda t, h: (h, 0)),
        ],
        out_specs=pl.BlockSpec((Sblk, D), lambda t, h: (t, 0)),
        out_shape=jax.ShapeDtypeStruct((S, D), F32),
    )(xb, routed, sgb, sub, sdb)

    return out.reshape(B, S, D)


# speedup vs baseline: 1.2103x; 1.1546x over previous
import functools

import jax
import jax.numpy as jnp
from jax import lax
from jax.experimental import pallas as pl
from jax.experimental.pallas import tpu as pltpu
from jax.experimental.pallas import tpu_sc as plsc

F32 = jnp.float32
BF16 = jnp.bfloat16
I32 = jnp.int32

BS = 256
NW = 32
NC = 2


def _router_body(xb_ref, wr_ref, base_ref, st_ref, posb_ref, tok_ref, pos_ref,
                 d12_ref, w12_ref, nb_ref, ps_ref, *, S, E, NB):
    logits = jax.lax.dot_general(
        xb_ref[...], wr_ref[...].astype(BF16),
        (((1,), (1,)), ((), ())), preferred_element_type=F32)
    scores = jax.nn.sigmoid(logits)
    tok = tok_ref[...]
    pos = pos_ref[...]
    bias_st = jnp.where(tok == 0, st_ref[0:1, :E], st_ref[1:2, :E])
    onehot = (jax.lax.broadcasted_iota(I32, (S, NB), 1) == pos).astype(F32)
    bias_pos = jnp.dot(onehot, posb_ref[...], preferred_element_type=F32,
                       precision=jax.lax.Precision.HIGHEST)
    gating = scores + base_ref[...] + bias_st + bias_pos

    lane = jax.lax.broadcasted_iota(I32, (S, E), 1)
    m1 = jnp.max(gating, axis=1, keepdims=True)
    idx1 = jnp.min(jnp.where(gating == m1, lane, E), axis=1, keepdims=True)
    g2 = jnp.where(lane == idx1, -jnp.inf, gating)
    m2 = jnp.max(g2, axis=1, keepdims=True)
    idx2 = jnp.min(jnp.where(g2 == m2, lane, E), axis=1, keepdims=True)
    s1 = jnp.sum(jnp.where(lane == idx1, scores, 0.0), axis=1, keepdims=True)
    s2 = jnp.sum(jnp.where(lane == idx2, scores, 0.0), axis=1, keepdims=True)
    denom = s1 + s2 + jnp.float32(1e-9)

    C = ((lane == idx1) | (lane == idx2)).astype(BF16)
    r_i = jax.lax.broadcasted_iota(I32, (S, S), 0)
    c_i = jax.lax.broadcasted_iota(I32, (S, S), 1)
    LT = (c_i < r_i).astype(BF16)
    EX = jax.lax.dot_general(LT, C, (((1,), (0,)), ((), ())),
                             preferred_element_type=F32)
    counts = jnp.sum(C.astype(F32), axis=0, keepdims=True)
    cnt_i = counts.astype(I32)
    nb = (cnt_i + (BS - 1)) // BS
    lane8r = jax.lax.broadcasted_iota(I32, (E, E), 0)
    lane8c = jax.lax.broadcasted_iota(I32, (E, E), 1)
    W8 = (lane8r < lane8c).astype(F32)
    ps = jnp.dot(nb.astype(F32), W8, preferred_element_type=F32,
                 precision=jax.lax.Precision.HIGHEST)
    ps_b = jnp.broadcast_to(ps, (S, E))
    rank1 = jnp.sum(jnp.where(lane == idx1, EX, 0.0), axis=1, keepdims=True)
    rank2 = jnp.sum(jnp.where(lane == idx2, EX, 0.0), axis=1, keepdims=True)
    base1 = jnp.sum(jnp.where(lane == idx1, ps_b, 0.0), axis=1, keepdims=True)
    base2 = jnp.sum(jnp.where(lane == idx2, ps_b, 0.0), axis=1, keepdims=True)
    dest1 = (base1 * BS + rank1).astype(I32)
    dest2 = (base2 * BS + rank2).astype(I32)

    two = jax.lax.broadcasted_iota(I32, (S, 2), 1)
    d12_ref[...] = jnp.where(two == 0, dest1, dest2)
    w12_ref[...] = jnp.where(two == 0, s1 / denom, s2 / denom)
    nb_ref[...] = nb
    ps_ref[...] = ps.astype(I32)


def _dispatch_body(x_hbm, tok3_hbm, dst3_hbm, wrow3_hbm, xs_hbm, ws_hbm,
                   tok_v, dst_v, wrow_v, rows_v, sem):
    wid = lax.axis_index("s") * NC + lax.axis_index("c")
    pltpu.sync_copy(tok3_hbm.at[wid], tok_v)
    pltpu.sync_copy(dst3_hbm.at[wid], dst_v)
    pltpu.sync_copy(wrow3_hbm.at[wid], wrow_v)

    def chunk(c, carry):
        pltpu.async_copy(x_hbm.at[tok_v.at[c]], rows_v, sem).wait()
        pltpu.async_copy(rows_v, xs_hbm.at[dst_v.at[c]], sem).wait()
        pltpu.async_copy(wrow_v.at[c], ws_hbm.at[dst_v.at[c]], sem).wait()
        return carry

    lax.fori_loop(0, 8, chunk, 0)


def _grouped_body(nb_ref, ps_ref, xs_ref, ws_ref, g_ref, u_ref, d_ref,
                  ys_ref, *, E):
    e = pl.program_id(0)
    i = pl.program_id(1)

    @pl.when(i < nb_ref[e])
    def _():
        xb = xs_ref[...].astype(BF16)
        g = jnp.dot(xb, g_ref[0], preferred_element_type=F32)
        u = jnp.dot(xb, u_ref[0], preferred_element_type=F32)
        c = ws_ref[:, 0:1]
        hc = ((jax.nn.silu(g) * u) * c).astype(BF16)
        ys_ref[...] = jnp.dot(hc, d_ref[0], preferred_element_type=F32)


def _shared_body(xb_ref, g_ref, u_ref, d_ref, out_ref):
    hb = pl.program_id(1)
    xb = xb_ref[...]
    g = jnp.dot(xb, g_ref[...], preferred_element_type=F32)
    u = jnp.dot(xb, u_ref[...], preferred_element_type=F32)
    h = (jax.nn.silu(g) * u).astype(BF16)
    y = jnp.dot(h, d_ref[...], preferred_element_type=F32)

    @pl.when(hb == 0)
    def _():
        out_ref[...] = y

    @pl.when(hb != 0)
    def _():
        out_ref[...] += y


def _combine_body(sh_hbm, ys_hbm, d1_hbm, d2_hbm, out_hbm,
                  acc_v, y1_v, y2_v, d1_v, d2_v, s0, s1, s2):
    wid = lax.axis_index("s") * NC + lax.axis_index("c")
    t0 = wid * 64
    pltpu.sync_copy(d1_hbm.at[wid], d1_v)
    pltpu.sync_copy(d2_hbm.at[wid], d2_v)

    def chunk(c, carry):
        base = t0 + c * 16
        cp0 = pltpu.async_copy(sh_hbm.at[pl.ds(base, 16)], acc_v, s0)
        cp1 = pltpu.async_copy(ys_hbm.at[d1_v.at[c]], y1_v, s1)
        cp2 = pltpu.async_copy(ys_hbm.at[d2_v.at[c]], y2_v, s2)
        cp0.wait()
        cp1.wait()
        cp2.wait()
        for r in range(16):
            def col(j2, carry2):
                for k in range(8):
                    sl = pl.ds(j2 * 128 + k * 16, 16)
                    acc_v[r, sl] = acc_v[r, sl] + y1_v[r, sl] + y2_v[r, sl]
                return carry2
            lax.fori_loop(0, 16, col, 0)
        pltpu.sync_copy(acc_v, out_hbm.at[pl.ds(base, 16)])
        return carry

    lax.fori_loop(0, 4, chunk, 0)


def _dispatch_sc(x, tok3, dst3, wrow3, CAP, D):
    mesh = plsc.VectorSubcoreMesh(core_axis_name="c", subcore_axis_name="s")
    f = pl.kernel(
        _dispatch_body, mesh=mesh,
        out_type=(jax.ShapeDtypeStruct((CAP, D), F32),
                  jax.ShapeDtypeStruct((CAP, 128), F32)),
        scratch_types=[
            pltpu.VMEM((8, 16), I32),
            pltpu.VMEM((8, 16), I32),
            pltpu.VMEM((8, 16, 128), F32),
            pltpu.VMEM((16, D), F32),
            pltpu.SemaphoreType.DMA,
        ],
    )
    return f(x, tok3, dst3, wrow3)


def _combine_sc(shared, ys, d1_3, d2_3, S, D):
    mesh = plsc.VectorSubcoreMesh(core_axis_name="c", subcore_axis_name="s")
    f = pl.kernel(
        _combine_body, mesh=mesh,
        out_type=jax.ShapeDtypeStruct((S, D), F32),
        scratch_types=[
            pltpu.VMEM((16, D), F32),
            pltpu.VMEM((16, D), F32),
            pltpu.VMEM((16, D), F32),
            pltpu.VMEM((4, 16), I32),
            pltpu.VMEM((4, 16), I32),
            pltpu.SemaphoreType.DMA,
            pltpu.SemaphoreType.DMA,
            pltpu.SemaphoreType.DMA,
        ],
    )
    return f(shared, ys, d1_3, d2_3)


def kernel(hidden_states, diffusion_timestep, diffusion_token_state,
           position_ids, W_router, ts_bias, st_bias, pos_bias, router_bias,
           routed_gate, routed_up, routed_down,
           shared_gate, shared_up, shared_down):
    B, S, D = hidden_states.shape
    E, _, H = routed_gate.shape
    NB = pos_bias.shape[0]
    Hs = shared_gate.shape[1]
    CAP = S * 2 + E * BS

    x = hidden_states.reshape(S, D)
    xb = x.astype(BF16)
    tok = diffusion_token_state.reshape(S, 1).astype(I32)
    pos = jnp.clip(position_ids.reshape(S, 1).astype(I32), 0, NB - 1)
    base = (router_bias + ts_bias[diffusion_timestep[0]]).reshape(1, E)
    st_pad = jnp.zeros((8, E), F32).at[:2].set(st_bias)

    d12, w12, nb, ps = pl.pallas_call(
        functools.partial(_router_body, S=S, E=E, NB=NB),
        out_shape=(jax.ShapeDtypeStruct((S, 2), I32),
                   jax.ShapeDtypeStruct((S, 2), F32),
                   jax.ShapeDtypeStruct((1, E), I32),
                   jax.ShapeDtypeStruct((1, E), I32)),
    )(xb, W_router, base, st_pad, pos_bias, tok, pos)

    tok_pair = jnp.broadcast_to(jnp.arange(S, dtype=I32)[:, None], (S, 2))
    tok3 = tok_pair.reshape(NW, 8, 16)
    dst3 = d12.reshape(NW, 8, 16)
    wrow3 = jnp.broadcast_to(w12.reshape(S * 2, 1), (S * 2, 128)
                             ).reshape(NW, 8, 16, 128)

    xs, ws16 = _dispatch_sc(x, tok3, dst3, wrow3, CAP, D)

    NBmax = S // BS
    gb = routed_gate.astype(BF16)
    ub = routed_up.astype(BF16)
    db = routed_down.astype(BF16)
    ys = pl.pallas_call(
        functools.partial(_grouped_body, E=E),
        grid_spec=pltpu.PrefetchScalarGridSpec(
            num_scalar_prefetch=2,
            grid=(E, NBmax),
            in_specs=[
                pl.BlockSpec((BS, D), lambda e, i, nb_, ps_: (
                    ps_[e] + jnp.minimum(i, jnp.maximum(nb_[e] - 1, 0)), 0)),
                pl.BlockSpec((BS, 128), lambda e, i, nb_, ps_: (
                    ps_[e] + jnp.minimum(i, jnp.maximum(nb_[e] - 1, 0)), 0)),
                pl.BlockSpec((1, D, H), lambda e, i, nb_, ps_: (e, 0, 0)),
                pl.BlockSpec((1, D, H), lambda e, i, nb_, ps_: (e, 0, 0)),
                pl.BlockSpec((1, H, D), lambda e, i, nb_, ps_: (e, 0, 0)),
            ],
            out_specs=pl.BlockSpec((BS, D), lambda e, i, nb_, ps_: (
                ps_[e] + jnp.minimum(i, jnp.maximum(nb_[e] - 1, 0)), 0)),
        ),
        out_shape=jax.ShapeDtypeStruct((CAP, D), F32),
    )(nb.reshape(E), ps.reshape(E), xs, ws16, gb, ub, db)

    TB = 4
    Sblk = S // TB
    HB = 4
    Hblk = Hs // HB
    sgb = shared_gate.astype(BF16)
    sub = shared_up.astype(BF16)
    sdb = shared_down.astype(BF16)
    shared = pl.pallas_call(
        _shared_body,
        grid=(TB, HB),
        in_specs=[
            pl.BlockSpec((Sblk, D), lambda t, h: (t, 0)),
            pl.BlockSpec((D, Hblk), lambda t, h: (0, h)),
            pl.BlockSpec((D, Hblk), lambda t, h: (0, h)),
            pl.BlockSpec((Hblk, D), lambda t, h: (h, 0)),
        ],
        out_specs=pl.BlockSpec((Sblk, D), lambda t, h: (t, 0)),
        out_shape=jax.ShapeDtypeStruct((S, D), F32),
    )(xb, sgb, sub, sdb)

    d1_3 = d12[:, 0].reshape(NW, 4, 16)
    d2_3 = d12[:, 1].reshape(NW, 4, 16)
    out = _combine_sc(shared, ys, d1_3, d2_3, S, D)

    return out.reshape(B, S, D)


# speedup vs baseline: 1.3146x; 1.0862x over previous
import functools

import jax
import jax.numpy as jnp
from jax import lax
from jax.experimental import pallas as pl
from jax.experimental.pallas import tpu as pltpu
from jax.experimental.pallas import tpu_sc as plsc

F32 = jnp.float32
BF16 = jnp.bfloat16
I32 = jnp.int32

BS = 256
NW = 32
NC = 2


def _router_body(xb_ref, wr_ref, base_ref, st_ref, posb_ref, tok_ref, pos_ref,
                 d12_ref, w12_ref, nb_ref, ps_ref, be_ref, bm_ref,
                 *, S, E, NB):
    logits = jax.lax.dot_general(
        xb_ref[...], wr_ref[...].astype(BF16),
        (((1,), (1,)), ((), ())), preferred_element_type=F32)
    scores = jax.nn.sigmoid(logits)
    tok = tok_ref[...]
    pos = pos_ref[...]
    bias_st = jnp.where(tok == 0, st_ref[0:1, :E], st_ref[1:2, :E])
    onehot = (jax.lax.broadcasted_iota(I32, (S, NB), 1) == pos).astype(F32)
    bias_pos = jnp.dot(onehot, posb_ref[...], preferred_element_type=F32,
                       precision=jax.lax.Precision.HIGHEST)
    gating = scores + base_ref[...] + bias_st + bias_pos

    lane = jax.lax.broadcasted_iota(I32, (S, E), 1)
    m1 = jnp.max(gating, axis=1, keepdims=True)
    idx1 = jnp.min(jnp.where(gating == m1, lane, E), axis=1, keepdims=True)
    g2 = jnp.where(lane == idx1, -jnp.inf, gating)
    m2 = jnp.max(g2, axis=1, keepdims=True)
    idx2 = jnp.min(jnp.where(g2 == m2, lane, E), axis=1, keepdims=True)
    s1 = jnp.sum(jnp.where(lane == idx1, scores, 0.0), axis=1, keepdims=True)
    s2 = jnp.sum(jnp.where(lane == idx2, scores, 0.0), axis=1, keepdims=True)
    denom = s1 + s2 + jnp.float32(1e-9)

    C = ((lane == idx1) | (lane == idx2)).astype(BF16)
    r_i = jax.lax.broadcasted_iota(I32, (S, S), 0)
    c_i = jax.lax.broadcasted_iota(I32, (S, S), 1)
    LT = (c_i < r_i).astype(BF16)
    EX = jax.lax.dot_general(LT, C, (((1,), (0,)), ((), ())),
                             preferred_element_type=F32)
    counts = jnp.sum(C.astype(F32), axis=0, keepdims=True)
    cnt_i = counts.astype(I32)
    nb = (cnt_i + (BS - 1)) // BS
    lane8r = jax.lax.broadcasted_iota(I32, (E, E), 0)
    lane8c = jax.lax.broadcasted_iota(I32, (E, E), 1)
    W8 = (lane8r < lane8c).astype(F32)
    ps = jnp.dot(nb.astype(F32), W8, preferred_element_type=F32,
                 precision=jax.lax.Precision.HIGHEST)
    ps_b = jnp.broadcast_to(ps, (S, E))
    rank1 = jnp.sum(jnp.where(lane == idx1, EX, 0.0), axis=1, keepdims=True)
    rank2 = jnp.sum(jnp.where(lane == idx2, EX, 0.0), axis=1, keepdims=True)
    base1 = jnp.sum(jnp.where(lane == idx1, ps_b, 0.0), axis=1, keepdims=True)
    base2 = jnp.sum(jnp.where(lane == idx2, ps_b, 0.0), axis=1, keepdims=True)
    dest1 = (base1 * BS + rank1).astype(I32)
    dest2 = (base2 * BS + rank2).astype(I32)

    two = jax.lax.broadcasted_iota(I32, (S, 2), 1)
    d12_ref[...] = jnp.where(two == 0, dest1, dest2)
    w12_ref[...] = jnp.where(two == 0, s1 / denom, s2 / denom)
    nb_ref[...] = nb
    ps_ref[...] = ps.astype(I32)

    psi = ps.astype(I32)
    biota = jax.lax.broadcasted_iota(I32, (32, E), 0)
    be_ref[...] = (jnp.sum((psi <= biota).astype(I32), axis=1, keepdims=True)
                   - 1)
    tot = jnp.sum(nb, axis=1, keepdims=True)
    bcol = jax.lax.broadcasted_iota(I32, (32, 1), 0)
    bm_ref[...] = jnp.minimum(bcol, tot - 1)


def _dispatch_body(x_hbm, tok3_hbm, dst3_hbm, wrow3_hbm, xs_hbm, ws_hbm,
                   tok_v, dst_v, wrow_v, r0, r1, sg0, sg1, ss0, ss1, sw):
    wid = lax.axis_index("s") * NC + lax.axis_index("c")
    pltpu.sync_copy(tok3_hbm.at[wid], tok_v)
    pltpu.sync_copy(dst3_hbm.at[wid], dst_v)
    pltpu.sync_copy(wrow3_hbm.at[wid], wrow_v)

    rows = [r0, r1]
    sgs = [sg0, sg1]
    sss = [ss0, ss1]
    wcopies = [pltpu.async_copy(wrow_v.at[c], ws_hbm.at[dst_v.at[c]], sw)
               for c in range(8)]
    sc_pend = [None, None]
    g = pltpu.async_copy(x_hbm.at[tok_v.at[0]], rows[0], sgs[0])
    for c in range(8):
        b = c & 1
        o = 1 - b
        g.wait()
        if c < 7:
            if sc_pend[o] is not None:
                sc_pend[o].wait()
            g = pltpu.async_copy(x_hbm.at[tok_v.at[c + 1]], rows[o], sgs[o])
        sc_pend[b] = pltpu.async_copy(rows[b], xs_hbm.at[dst_v.at[c]], sss[b])
    sc_pend[0].wait()
    sc_pend[1].wait()
    for w in wcopies:
        w.wait()


def _grouped_body(be_ref, bm_ref, xs_ref, ws_ref, g_ref, u_ref, d_ref,
                  ys_ref, *, E):
    b = pl.program_id(0)

    @pl.when(bm_ref[b] == b)
    def _():
        xb = xs_ref[...].astype(BF16)
        g = jnp.dot(xb, g_ref[0], preferred_element_type=F32)
        u = jnp.dot(xb, u_ref[0], preferred_element_type=F32)
        c = ws_ref[:, 0:1]
        hc = ((jax.nn.silu(g) * u) * c).astype(BF16)
        ys_ref[...] = jnp.dot(hc, d_ref[0], preferred_element_type=F32)


def _shared_body(xb_ref, g_ref, u_ref, d_ref, out_ref):
    hb = pl.program_id(1)
    xb = xb_ref[...]
    g = jnp.dot(xb, g_ref[...], preferred_element_type=F32)
    u = jnp.dot(xb, u_ref[...], preferred_element_type=F32)
    h = (jax.nn.silu(g) * u).astype(BF16)
    y = jnp.dot(h, d_ref[...], preferred_element_type=F32)

    @pl.when(hb == 0)
    def _():
        out_ref[...] = y

    @pl.when(hb != 0)
    def _():
        out_ref[...] += y


def _combine_body(sh_hbm, ys_hbm, d1_hbm, d2_hbm, out_hbm,
                  a0, a1, y10, y11, y20, y21, d1_v, d2_v,
                  sa0, sa1, s10, s11, s20, s21):
    wid = lax.axis_index("s") * NC + lax.axis_index("c")
    t0 = wid * 64
    pltpu.sync_copy(d1_hbm.at[wid], d1_v)
    pltpu.sync_copy(d2_hbm.at[wid], d2_v)

    accs = [a0, a1]
    y1s = [y10, y11]
    y2s = [y20, y21]
    sas = [sa0, sa1]
    s1s = [s10, s11]
    s2s = [s20, s21]

    def fire(c, b):
        base = t0 + c * 8
        return (pltpu.async_copy(sh_hbm.at[pl.ds(base, 8)], accs[b], sas[b]),
                pltpu.async_copy(ys_hbm.at[d1_v.at[c]], y1s[b], s1s[b]),
                pltpu.async_copy(ys_hbm.at[d2_v.at[c]], y2s[b], s2s[b]))

    cps = fire(0, 0)
    for c in range(8):
        b = c & 1
        for cp in cps:
            cp.wait()
        if c < 7:
            cps = fire(c + 1, 1 - b)
        for r in range(8):
            def col(j2, carry2, _r=r, _b=b):
                for k in range(8):
                    sl = pl.ds(j2 * 128 + k * 16, 16)
                    accs[_b][_r, sl] = (accs[_b][_r, sl]
                                        + y1s[_b][_r, sl] + y2s[_b][_r, sl])
                return carry2
            lax.fori_loop(0, 16, col, 0)
        pltpu.sync_copy(accs[b], out_hbm.at[pl.ds(t0 + c * 8, 8)])


def _dispatch_sc(x, tok3, dst3, wrow3, CAP, D):
    mesh = plsc.VectorSubcoreMesh(core_axis_name="c", subcore_axis_name="s")
    f = pl.kernel(
        _dispatch_body, mesh=mesh,
        out_type=(jax.ShapeDtypeStruct((CAP, D), F32),
                  jax.ShapeDtypeStruct((CAP, 128), F32)),
        scratch_types=[
            pltpu.VMEM((8, 16), I32),
            pltpu.VMEM((8, 16), I32),
            pltpu.VMEM((8, 16, 128), F32),
            pltpu.VMEM((16, D), F32),
            pltpu.VMEM((16, D), F32),
            pltpu.SemaphoreType.DMA,
            pltpu.SemaphoreType.DMA,
            pltpu.SemaphoreType.DMA,
            pltpu.SemaphoreType.DMA,
            pltpu.SemaphoreType.DMA,
        ],
    )
    return f(x, tok3, dst3, wrow3)


def _combine_sc(shared, ys, d1_3, d2_3, S, D):
    mesh = plsc.VectorSubcoreMesh(core_axis_name="c", subcore_axis_name="s")
    f = pl.kernel(
        _combine_body, mesh=mesh,
        out_type=jax.ShapeDtypeStruct((S, D), F32),
        scratch_types=[
            pltpu.VMEM((8, D), F32),
            pltpu.VMEM((8, D), F32),
            pltpu.VMEM((8, D), F32),
            pltpu.VMEM((8, D), F32),
            pltpu.VMEM((8, D), F32),
            pltpu.VMEM((8, D), F32),
            pltpu.VMEM((8, 8), I32),
            pltpu.VMEM((8, 8), I32),
            pltpu.SemaphoreType.DMA,
            pltpu.SemaphoreType.DMA,
            pltpu.SemaphoreType.DMA,
            pltpu.SemaphoreType.DMA,
            pltpu.SemaphoreType.DMA,
            pltpu.SemaphoreType.DMA,
        ],
    )
    return f(shared, ys, d1_3, d2_3)


def kernel(hidden_states, diffusion_timestep, diffusion_token_state,
           position_ids, W_router, ts_bias, st_bias, pos_bias, router_bias,
           routed_gate, routed_up, routed_down,
           shared_gate, shared_up, shared_down):
    B, S, D = hidden_states.shape
    E, _, H = routed_gate.shape
    NB = pos_bias.shape[0]
    Hs = shared_gate.shape[1]
    CAP = S * 2 + E * BS

    x = hidden_states.reshape(S, D)
    xb = x.astype(BF16)
    tok = diffusion_token_state.reshape(S, 1).astype(I32)
    pos = jnp.clip(position_ids.reshape(S, 1).astype(I32), 0, NB - 1)
    base = (router_bias + ts_bias[diffusion_timestep[0]]).reshape(1, E)
    st_pad = jnp.zeros((8, E), F32).at[:2].set(st_bias)

    d12, w12, nb, ps, be32, bm32 = pl.pallas_call(
        functools.partial(_router_body, S=S, E=E, NB=NB),
        out_shape=(jax.ShapeDtypeStruct((S, 2), I32),
                   jax.ShapeDtypeStruct((S, 2), F32),
                   jax.ShapeDtypeStruct((1, E), I32),
                   jax.ShapeDtypeStruct((1, E), I32),
                   jax.ShapeDtypeStruct((32, 1), I32),
                   jax.ShapeDtypeStruct((32, 1), I32)),
    )(xb, W_router, base, st_pad, pos_bias, tok, pos)

    tok_pair = jnp.broadcast_to(jnp.arange(S, dtype=I32)[:, None], (S, 2))
    tok3 = tok_pair.reshape(NW, 8, 16)
    dst3 = d12.reshape(NW, 8, 16)
    wrow3 = jnp.broadcast_to(w12.reshape(S * 2, 1), (S * 2, 128)
                             ).reshape(NW, 8, 16, 128)

    xs, ws16 = _dispatch_sc(x, tok3, dst3, wrow3, CAP, D)

    TOTB = CAP // BS
    gb = routed_gate.astype(BF16)
    ub = routed_up.astype(BF16)
    db = routed_down.astype(BF16)
    ys = pl.pallas_call(
        functools.partial(_grouped_body, E=E),
        grid_spec=pltpu.PrefetchScalarGridSpec(
            num_scalar_prefetch=2,
            grid=(TOTB,),
            in_specs=[
                pl.BlockSpec((BS, D), lambda b, be_, bm_: (bm_[b], 0)),
                pl.BlockSpec((BS, 128), lambda b, be_, bm_: (bm_[b], 0)),
                pl.BlockSpec((1, D, H), lambda b, be_, bm_: (be_[b], 0, 0)),
                pl.BlockSpec((1, D, H), lambda b, be_, bm_: (be_[b], 0, 0)),
                pl.BlockSpec((1, H, D), lambda b, be_, bm_: (be_[b], 0, 0)),
            ],
            out_specs=pl.BlockSpec((BS, D), lambda b, be_, bm_: (bm_[b], 0)),
        ),
        out_shape=jax.ShapeDtypeStruct((CAP, D), F32),
    )(be32.reshape(32), bm32.reshape(32), xs, ws16, gb, ub, db)

    TB = 4
    Sblk = S // TB
    HB = 4
    Hblk = Hs // HB
    sgb = shared_gate.astype(BF16)
    sub = shared_up.astype(BF16)
    sdb = shared_down.astype(BF16)
    shared = pl.pallas_call(
        _shared_body,
        grid=(TB, HB),
        in_specs=[
            pl.BlockSpec((Sblk, D), lambda t, h: (t, 0)),
            pl.BlockSpec((D, Hblk), lambda t, h: (0, h)),
            pl.BlockSpec((D, Hblk), lambda t, h: (0, h)),
            pl.BlockSpec((Hblk, D), lambda t, h: (h, 0)),
        ],
        out_specs=pl.BlockSpec((Sblk, D), lambda t, h: (t, 0)),
        out_shape=jax.ShapeDtypeStruct((S, D), F32),
    )(xb, sgb, sub, sdb)

    d1_3 = d12[:, 0].reshape(NW, 8, 8)
    d2_3 = d12[:, 1].reshape(NW, 8, 8)
    out = _combine_sc(shared, ys, d1_3, d2_3, S, D)

    return out.reshape(B, S, D)


# speedup vs baseline: 1.3177x; 1.0023x over previous
import functools

import jax
import jax.numpy as jnp
from jax import lax
from jax.experimental import pallas as pl
from jax.experimental.pallas import tpu as pltpu
from jax.experimental.pallas import tpu_sc as plsc

F32 = jnp.float32
BF16 = jnp.bfloat16
I32 = jnp.int32

BS = 256
NW = 32
NC = 2


def _router_body(xb_ref, wr_ref, base_ref, st_ref, posb_ref, tok_ref, pos_ref,
                 d12_ref, w12_ref, nb_ref, ps_ref, be_ref, bm_ref,
                 *, S, E, NB):
    logits = jax.lax.dot_general(
        xb_ref[...], wr_ref[...].astype(BF16),
        (((1,), (1,)), ((), ())), preferred_element_type=F32)
    scores = jax.nn.sigmoid(logits)
    tok = tok_ref[...]
    pos = pos_ref[...]
    bias_st = jnp.where(tok == 0, st_ref[0:1, :E], st_ref[1:2, :E])
    onehot = (jax.lax.broadcasted_iota(I32, (S, NB), 1) == pos).astype(F32)
    bias_pos = jnp.dot(onehot, posb_ref[...], preferred_element_type=F32,
                       precision=jax.lax.Precision.HIGHEST)
    gating = scores + base_ref[...] + bias_st + bias_pos

    lane = jax.lax.broadcasted_iota(I32, (S, E), 1)
    m1 = jnp.max(gating, axis=1, keepdims=True)
    idx1 = jnp.min(jnp.where(gating == m1, lane, E), axis=1, keepdims=True)
    g2 = jnp.where(lane == idx1, -jnp.inf, gating)
    m2 = jnp.max(g2, axis=1, keepdims=True)
    idx2 = jnp.min(jnp.where(g2 == m2, lane, E), axis=1, keepdims=True)
    s1 = jnp.sum(jnp.where(lane == idx1, scores, 0.0), axis=1, keepdims=True)
    s2 = jnp.sum(jnp.where(lane == idx2, scores, 0.0), axis=1, keepdims=True)
    denom = s1 + s2 + jnp.float32(1e-9)

    C = ((lane == idx1) | (lane == idx2)).astype(BF16)
    r_i = jax.lax.broadcasted_iota(I32, (S, S), 0)
    c_i = jax.lax.broadcasted_iota(I32, (S, S), 1)
    LT = (c_i < r_i).astype(BF16)
    EX = jax.lax.dot_general(LT, C, (((1,), (0,)), ((), ())),
                             preferred_element_type=F32)
    counts = jnp.sum(C.astype(F32), axis=0, keepdims=True)
    cnt_i = counts.astype(I32)
    nb = (cnt_i + (BS - 1)) // BS
    lane8r = jax.lax.broadcasted_iota(I32, (E, E), 0)
    lane8c = jax.lax.broadcasted_iota(I32, (E, E), 1)
    W8 = (lane8r < lane8c).astype(F32)
    ps = jnp.dot(nb.astype(F32), W8, preferred_element_type=F32,
                 precision=jax.lax.Precision.HIGHEST)
    ps_b = jnp.broadcast_to(ps, (S, E))
    rank1 = jnp.sum(jnp.where(lane == idx1, EX, 0.0), axis=1, keepdims=True)
    rank2 = jnp.sum(jnp.where(lane == idx2, EX, 0.0), axis=1, keepdims=True)
    base1 = jnp.sum(jnp.where(lane == idx1, ps_b, 0.0), axis=1, keepdims=True)
    base2 = jnp.sum(jnp.where(lane == idx2, ps_b, 0.0), axis=1, keepdims=True)
    dest1 = (base1 * BS + rank1).astype(I32)
    dest2 = (base2 * BS + rank2).astype(I32)

    two = jax.lax.broadcasted_iota(I32, (S, 2), 1)
    d12_ref[...] = jnp.where(two == 0, dest1, dest2)
    w12_ref[...] = jnp.where(two == 0, s1 / denom, s2 / denom)
    nb_ref[...] = nb
    ps_ref[...] = ps.astype(I32)

    psi = ps.astype(I32)
    biota = jax.lax.broadcasted_iota(I32, (32, E), 0)
    be_ref[...] = (jnp.sum((psi <= biota).astype(I32), axis=1, keepdims=True)
                   - 1)
    tot = jnp.sum(nb, axis=1, keepdims=True)
    bcol = jax.lax.broadcasted_iota(I32, (32, 1), 0)
    bm_ref[...] = jnp.minimum(bcol, tot - 1)


def _dispatch_body(x_hbm, tok3_hbm, dst3_hbm, wrow3_hbm, xs_hbm, ws_hbm,
                   tok_v, dst_v, wrow_v, r0, r1, sg0, sg1, ss0, ss1, sw):
    wid = lax.axis_index("s") * NC + lax.axis_index("c")
    pltpu.sync_copy(tok3_hbm.at[wid], tok_v)
    pltpu.sync_copy(dst3_hbm.at[wid], dst_v)
    pltpu.sync_copy(wrow3_hbm.at[wid], wrow_v)

    rows = [r0, r1]
    sgs = [sg0, sg1]
    sss = [ss0, ss1]
    wcopies = [pltpu.async_copy(wrow_v.at[c], ws_hbm.at[dst_v.at[c]], sw)
               for c in range(8)]
    sc_pend = [None, None]
    g = pltpu.async_copy(x_hbm.at[tok_v.at[0]], rows[0], sgs[0])
    for c in range(8):
        b = c & 1
        o = 1 - b
        g.wait()
        if c < 7:
            if sc_pend[o] is not None:
                sc_pend[o].wait()
            g = pltpu.async_copy(x_hbm.at[tok_v.at[c + 1]], rows[o], sgs[o])
        sc_pend[b] = pltpu.async_copy(rows[b], xs_hbm.at[dst_v.at[c]], sss[b])
    sc_pend[0].wait()
    sc_pend[1].wait()
    for w in wcopies:
        w.wait()


def _grouped_body(be_ref, bm_ref, xs_ref, ws_ref, g_ref, u_ref, d_ref,
                  ys_ref, *, E):
    b = pl.program_id(0)

    @pl.when(bm_ref[b] == b)
    def _():
        xb = xs_ref[...].astype(BF16)
        g = jnp.dot(xb, g_ref[0], preferred_element_type=F32)
        u = jnp.dot(xb, u_ref[0], preferred_element_type=F32)
        c = ws_ref[:, 0:1]
        hc = ((jax.nn.silu(g) * u) * c).astype(BF16)
        ys_ref[...] = jnp.dot(hc, d_ref[0], preferred_element_type=F32)


def _shared_body(xb_ref, g_ref, u_ref, d_ref, out_ref):
    hb = pl.program_id(1)
    xb = xb_ref[...]
    g = jnp.dot(xb, g_ref[...], preferred_element_type=F32)
    u = jnp.dot(xb, u_ref[...], preferred_element_type=F32)
    h = (jax.nn.silu(g) * u).astype(BF16)
    y = jnp.dot(h, d_ref[...], preferred_element_type=F32)

    @pl.when(hb == 0)
    def _():
        out_ref[...] = y

    @pl.when(hb != 0)
    def _():
        out_ref[...] += y


def _combine_body(sh_hbm, ys_hbm, d1_hbm, d2_hbm, out_hbm,
                  a0, a1, y10, y11, y20, y21, d1_v, d2_v,
                  sa0, sa1, s10, s11, s20, s21):
    wid = lax.axis_index("s") * NC + lax.axis_index("c")
    t0 = wid * 64
    pltpu.sync_copy(d1_hbm.at[wid], d1_v)
    pltpu.sync_copy(d2_hbm.at[wid], d2_v)

    accs = [a0, a1]
    y1s = [y10, y11]
    y2s = [y20, y21]
    sas = [sa0, sa1]
    s1s = [s10, s11]
    s2s = [s20, s21]

    def fire(c, b):
        base = t0 + c * 8
        return (pltpu.async_copy(sh_hbm.at[pl.ds(base, 8)], accs[b], sas[b]),
                pltpu.async_copy(ys_hbm.at[d1_v.at[c]], y1s[b], s1s[b]),
                pltpu.async_copy(ys_hbm.at[d2_v.at[c]], y2s[b], s2s[b]))

    cps = fire(0, 0)
    for c in range(8):
        b = c & 1
        for cp in cps:
            cp.wait()
        if c < 7:
            cps = fire(c + 1, 1 - b)
        for r in range(8):
            def col(j2, carry2, _r=r, _b=b):
                for k in range(8):
                    sl = pl.ds(j2 * 128 + k * 16, 16)
                    accs[_b][_r, sl] = (accs[_b][_r, sl]
                                        + y1s[_b][_r, sl] + y2s[_b][_r, sl])
                return carry2
            lax.fori_loop(0, 16, col, 0)
        pltpu.sync_copy(accs[b], out_hbm.at[pl.ds(t0 + c * 8, 8)])


def _dispatch_sc(x, tok3, dst3, wrow3, CAP, D):
    mesh = plsc.VectorSubcoreMesh(core_axis_name="c", subcore_axis_name="s")
    f = pl.kernel(
        _dispatch_body, mesh=mesh,
        out_type=(jax.ShapeDtypeStruct((CAP, D), F32),
                  jax.ShapeDtypeStruct((CAP, 128), F32)),
        scratch_types=[
            pltpu.VMEM((8, 16), I32),
            pltpu.VMEM((8, 16), I32),
            pltpu.VMEM((8, 16, 128), F32),
            pltpu.VMEM((16, D), F32),
            pltpu.VMEM((16, D), F32),
            pltpu.SemaphoreType.DMA,
            pltpu.SemaphoreType.DMA,
            pltpu.SemaphoreType.DMA,
            pltpu.SemaphoreType.DMA,
            pltpu.SemaphoreType.DMA,
        ],
    )
    return f(x, tok3, dst3, wrow3)


def _combine_sc(shared, ys, d1_3, d2_3, S, D):
    mesh = plsc.VectorSubcoreMesh(core_axis_name="c", subcore_axis_name="s")
    f = pl.kernel(
        _combine_body, mesh=mesh,
        out_type=jax.ShapeDtypeStruct((S, D), F32),
        scratch_types=[
            pltpu.VMEM((8, D), F32),
            pltpu.VMEM((8, D), F32),
            pltpu.VMEM((8, D), F32),
            pltpu.VMEM((8, D), F32),
            pltpu.VMEM((8, D), F32),
            pltpu.VMEM((8, D), F32),
            pltpu.VMEM((8, 8), I32),
            pltpu.VMEM((8, 8), I32),
            pltpu.SemaphoreType.DMA,
            pltpu.SemaphoreType.DMA,
            pltpu.SemaphoreType.DMA,
            pltpu.SemaphoreType.DMA,
            pltpu.SemaphoreType.DMA,
            pltpu.SemaphoreType.DMA,
        ],
    )
    return f(shared, ys, d1_3, d2_3)


def kernel(hidden_states, diffusion_timestep, diffusion_token_state,
           position_ids, W_router, ts_bias, st_bias, pos_bias, router_bias,
           routed_gate, routed_up, routed_down,
           shared_gate, shared_up, shared_down):
    B, S, D = hidden_states.shape
    E, _, H = routed_gate.shape
    NB = pos_bias.shape[0]
    Hs = shared_gate.shape[1]
    CAP = S * 2 + E * BS

    x = hidden_states.reshape(S, D)
    xb = x.astype(BF16)
    tok = diffusion_token_state.reshape(S, 1).astype(I32)
    pos = jnp.clip(position_ids.reshape(S, 1).astype(I32), 0, NB - 1)
    base = (router_bias + ts_bias[diffusion_timestep[0]]).reshape(1, E)
    st_pad = jnp.zeros((8, E), F32).at[:2].set(st_bias)

    d12, w12, nb, ps, be32, bm32 = pl.pallas_call(
        functools.partial(_router_body, S=S, E=E, NB=NB),
        out_shape=(jax.ShapeDtypeStruct((S, 2), I32),
                   jax.ShapeDtypeStruct((S, 2), F32),
                   jax.ShapeDtypeStruct((1, E), I32),
                   jax.ShapeDtypeStruct((1, E), I32),
                   jax.ShapeDtypeStruct((32, 1), I32),
                   jax.ShapeDtypeStruct((32, 1), I32)),
    )(xb, W_router, base, st_pad, pos_bias, tok, pos)

    tok_pair = jnp.broadcast_to(jnp.arange(S, dtype=I32)[:, None], (S, 2))
    tok3 = tok_pair.reshape(NW, 8, 16)
    dst3 = d12.reshape(NW, 8, 16)
    wrow3 = jnp.broadcast_to(w12.reshape(S * 2, 1), (S * 2, 128)
                             ).reshape(NW, 8, 16, 128)

    xs, ws16 = _dispatch_sc(x, tok3, dst3, wrow3, CAP, D)

    TB = 4
    Sblk = S // TB
    HB = 4
    Hblk = Hs // HB
    sgb = shared_gate.astype(BF16)
    sub = shared_up.astype(BF16)
    sdb = shared_down.astype(BF16)
    shared = pl.pallas_call(
        _shared_body,
        grid=(TB, HB),
        in_specs=[
            pl.BlockSpec((Sblk, D), lambda t, h: (t, 0)),
            pl.BlockSpec((D, Hblk), lambda t, h: (0, h)),
            pl.BlockSpec((D, Hblk), lambda t, h: (0, h)),
            pl.BlockSpec((Hblk, D), lambda t, h: (h, 0)),
        ],
        out_specs=pl.BlockSpec((Sblk, D), lambda t, h: (t, 0)),
        out_shape=jax.ShapeDtypeStruct((S, D), F32),
    )(xb, sgb, sub, sdb)

    TOTB = CAP // BS
    gb = routed_gate.astype(BF16)
    ub = routed_up.astype(BF16)
    db = routed_down.astype(BF16)
    ys = pl.pallas_call(
        functools.partial(_grouped_body, E=E),
        grid_spec=pltpu.PrefetchScalarGridSpec(
            num_scalar_prefetch=2,
            grid=(TOTB,),
            in_specs=[
                pl.BlockSpec((BS, D), lambda b, be_, bm_: (bm_[b], 0)),
                pl.BlockSpec((BS, 128), lambda b, be_, bm_: (bm_[b], 0)),
                pl.BlockSpec((1, D, H), lambda b, be_, bm_: (be_[b], 0, 0)),
                pl.BlockSpec((1, D, H), lambda b, be_, bm_: (be_[b], 0, 0)),
                pl.BlockSpec((1, H, D), lambda b, be_, bm_: (be_[b], 0, 0)),
            ],
            out_specs=pl.BlockSpec((BS, D), lambda b, be_, bm_: (bm_[b], 0)),
        ),
        out_shape=jax.ShapeDtypeStruct((CAP, D), F32),
    )(be32.reshape(32), bm32.reshape(32), xs, ws16, gb, ub, db)

    d1_3 = d12[:, 0].reshape(NW, 8, 8)
    d2_3 = d12[:, 1].reshape(NW, 8, 8)
    out = _combine_sc(shared, ys, d1_3, d2_3, S, D)

    return out.reshape(B, S, D)
